# Initial kernel scaffold; baseline (speedup 1.0000x reference)
import jax
import jax.numpy as jnp
from jax.experimental import pallas as pl


def kernel(x, edge_index, edge_attr, batch, scales, orientation, nm_W1, nm_b1, nm_W2, nm_b2, em_W1, em_b1, em_W2, em_b2, g1_W, g1_b, nn_W1, nn_b1, nn_W2, nn_b2, root_W, root_b, g2_W, g2_b, gate_W1, gate_b1, gate_W2, gate_b2, m_W1, m_b1, m_W2, m_b2, m_W3, m_b3):
    raise NotImplementedError("write your pallas kernel here")


# speedup vs baseline: 3.9882x; 3.9882x over previous
import functools

import jax
import jax.numpy as jnp
from jax import lax
from jax.experimental import pallas as pl
from jax.experimental.pallas import tpu as pltpu
from jax.experimental.pallas import tpu_sc as plsc

f32 = jnp.float32
i32 = jnp.int32

N = 10000
E = 160000
EP = 163840
CH = 128
NC = 2
NS = 16
NW = NC * NS
NP = 10240
HALF = N // 2
HP = HALF + 8

_mesh = lambda: plsc.VectorSubcoreMesh(core_axis_name="c", subcore_axis_name="s")


def _sc_count(dst_p, ones_rows, zeros_acc):
    @functools.partial(
        pl.kernel,
        out_type=jax.ShapeDtypeStruct((NC, NP, 16), f32),
        mesh=_mesh(),
        compiler_params=pltpu.CompilerParams(use_tc_tiling_on_sc=False),
        scratch_types=[
            pltpu.VMEM((1, CH), i32),
            pltpu.VMEM((CH, 16), f32),
            pltpu.VMEM_SHARED((NC, NP, 16), f32),
        ],
    )
    def k(dst_hbm, ones_hbm, zeros_hbm, out_hbm, idx_v, ones_v, acc):
        c = lax.axis_index("c")
        s = lax.axis_index("s")
        r0 = NP // NS
        pltpu.sync_copy(zeros_hbm.at[pl.ds(s * r0, r0)],
                        acc.at[c].at[pl.ds(s * r0, r0)])
        pltpu.sync_copy(ones_hbm, ones_v)
        plsc.subcore_barrier()
        wid = c * NS + s
        ebase = wid * (EP // NW)

        def chunk(ci, carry):
            base = ebase + ci * CH
            pltpu.sync_copy(dst_hbm.at[pl.ds(base, CH)], idx_v.at[0])
            pltpu.sync_copy(ones_v, acc.at[c].at[idx_v.at[0]], add=True)
            return carry

        lax.fori_loop(0, (EP // NW) // CH, chunk, 0)
        plsc.subcore_barrier()
        pltpu.sync_copy(acc.at[c].at[pl.ds(s * r0, r0)],
                        out_hbm.at[c].at[pl.ds(s * r0, r0)])

    return k(dst_p, ones_rows, zeros_acc)


def _sc_gather_scatter(src_p, dst_p, table, zeros_acc):
    @functools.partial(
        pl.kernel,
        out_type=jax.ShapeDtypeStruct((NC, NP, 16), f32),
        mesh=_mesh(),
        compiler_params=pltpu.CompilerParams(use_tc_tiling_on_sc=False),
        scratch_types=[
            pltpu.VMEM((1, CH), i32),
            pltpu.VMEM((1, CH), i32),
            pltpu.VMEM((CH, 16), f32),
            pltpu.VMEM_SHARED((NC, NP, 16), f32),
        ],
    )
    def k(src_hbm, dst_hbm, tab_hbm, zeros_hbm, out_hbm,
          sidx, didx, rows_v, acc):
        c = lax.axis_index("c")
        s = lax.axis_index("s")
        r0 = NP // NS
        pltpu.sync_copy(zeros_hbm.at[pl.ds(s * r0, r0)],
                        acc.at[c].at[pl.ds(s * r0, r0)])
        plsc.subcore_barrier()
        wid = c * NS + s
        ebase = wid * (EP // NW)

        def chunk(ci, carry):
            base = ebase + ci * CH
            pltpu.sync_copy(src_hbm.at[pl.ds(base, CH)], sidx.at[0])
            pltpu.sync_copy(dst_hbm.at[pl.ds(base, CH)], didx.at[0])
            pltpu.sync_copy(tab_hbm.at[sidx.at[0]], rows_v)
            pltpu.sync_copy(rows_v, acc.at[c].at[didx.at[0]], add=True)
            return carry

        lax.fori_loop(0, (EP // NW) // CH, chunk, 0)
        plsc.subcore_barrier()
        pltpu.sync_copy(acc.at[c].at[pl.ds(s * r0, r0)],
                        out_hbm.at[c].at[pl.ds(s * r0, r0)])

    return k(src_p, dst_p, table, zeros_acc)


def _sc_scatter_rows(dst_p, rows, zeros_acc):
    @functools.partial(
        pl.kernel,
        out_type=jax.ShapeDtypeStruct((NC, NP, 16), f32),
        mesh=_mesh(),
        compiler_params=pltpu.CompilerParams(use_tc_tiling_on_sc=False),
        scratch_types=[
            pltpu.VMEM((1, CH), i32),
            pltpu.VMEM((CH, 16), f32),
            pltpu.VMEM_SHARED((NC, NP, 16), f32),
        ],
    )
    def k(dst_hbm, rows_hbm, zeros_hbm, out_hbm, didx, rows_v, acc):
        c = lax.axis_index("c")
        s = lax.axis_index("s")
        r0 = NP // NS
        pltpu.sync_copy(zeros_hbm.at[pl.ds(s * r0, r0)],
                        acc.at[c].at[pl.ds(s * r0, r0)])
        plsc.subcore_barrier()
        wid = c * NS + s
        ebase = wid * (EP // NW)

        def chunk(ci, carry):
            base = ebase + ci * CH
            pltpu.sync_copy(rows_hbm.at[pl.ds(base, CH)], rows_v)
            pltpu.sync_copy(dst_hbm.at[pl.ds(base, CH)], didx.at[0])
            pltpu.sync_copy(rows_v, acc.at[c].at[didx.at[0]], add=True)
            return carry

        lax.fori_loop(0, (EP // NW) // CH, chunk, 0)
        plsc.subcore_barrier()
        pltpu.sync_copy(acc.at[c].at[pl.ds(s * r0, r0)],
                        out_hbm.at[c].at[pl.ds(s * r0, r0)])

    return k(dst_p, rows, zeros_acc)


def _sc_gather_rows(src_p, table):
    @functools.partial(
        pl.kernel,
        out_type=jax.ShapeDtypeStruct((EP, 16), f32),
        mesh=_mesh(),
        compiler_params=pltpu.CompilerParams(use_tc_tiling_on_sc=False),
        scratch_types=[
            pltpu.VMEM((1, CH), i32),
            pltpu.VMEM((CH, 16), f32),
        ],
    )
    def k(src_hbm, tab_hbm, out_hbm, sidx, rows_v):
        c = lax.axis_index("c")
        s = lax.axis_index("s")
        wid = c * NS + s
        ebase = wid * (EP // NW)

        def chunk(ci, carry):
            base = ebase + ci * CH
            pltpu.sync_copy(src_hbm.at[pl.ds(base, CH)], sidx.at[0])
            pltpu.sync_copy(tab_hbm.at[sidx.at[0]], rows_v)
            pltpu.sync_copy(rows_v, out_hbm.at[pl.ds(base, CH)])
            return carry

        lax.fori_loop(0, (EP // NW) // CH, chunk, 0)

    return k(src_p, table)


def _sc_scatter_max(src_p, dst_p, table, neg_rows):
    @functools.partial(
        pl.kernel,
        out_type=jax.ShapeDtypeStruct((NC, NS, HP, 16), f32),
        mesh=_mesh(),
        compiler_params=pltpu.CompilerParams(use_tc_tiling_on_sc=False),
        scratch_types=[
            pltpu.VMEM((1, CH), i32),
            pltpu.VMEM((1, CH), i32),
            pltpu.VMEM((CH, 16), f32),
            pltpu.VMEM((HP, 16), f32),
        ],
    )
    def k(src_hbm, dst_hbm, tab_hbm, neg_hbm, out_hbm,
          sidx, didx, rows_v, acc_v):
        c = lax.axis_index("c")
        s = lax.axis_index("s")
        pltpu.sync_copy(neg_hbm, acc_v)
        ebase = s * (EP // NS)
        nbase = c * HALF

        def chunk(ci, carry):
            base = ebase + ci * CH
            pltpu.sync_copy(src_hbm.at[pl.ds(base, CH)], sidx.at[0])
            pltpu.sync_copy(dst_hbm.at[pl.ds(base, CH)], didx.at[0])
            pltpu.sync_copy(tab_hbm.at[sidx.at[0]], rows_v)
            for g in range(CH // 16):
                dv = didx[0, pl.ds(g * 16, 16)]
                dl = dv - nbase
                ok = (dl >= 0) & (dl < HALF)
                idx16 = jnp.where(ok, dl, HALF)
                for l in range(16):
                    ri = idx16[l]
                    row = rows_v[g * 16 + l, :]
                    acc_v[ri, :] = jnp.maximum(acc_v[ri, :], row)
            return carry

        lax.fori_loop(0, (EP // NS) // CH, chunk, 0)
        pltpu.sync_copy(acc_v, out_hbm.at[c].at[s])

    return k(src_p, dst_p, table, neg_rows)


def _tc_prep(x, cnt0, cnt1, nm_W1, nm_b1, nm_W2, nm_b2, g1_W):
    BN = 1000
    grid = N // BN

    def body(x_ref, c0_ref, c1_ref, w1_ref, b1_ref, w2_ref, b2_ref, g1_ref,
             y_ref, xw_ref, dinv_ref, cnt_ref):
        xb = x_ref[...]
        h = jax.nn.relu(
            jnp.dot(xb, w1_ref[...], preferred_element_type=f32) + b1_ref[...])
        h = jnp.dot(h, w2_ref[...], preferred_element_type=f32) + b2_ref[...]
        xw = jnp.dot(h, g1_ref[...], preferred_element_type=f32)
        cnt = c0_ref[...] + c1_ref[...]
        dinv = lax.rsqrt(cnt + 2.0)
        y_ref[...] = dinv * xw
        xw_ref[...] = xw
        dinv_ref[...] = dinv
        cnt_ref[...] = cnt

    outs = pl.pallas_call(
        body,
        grid=(grid,),
        in_specs=[
            pl.BlockSpec((BN, 128), lambda i: (i, 0)),
            pl.BlockSpec((BN, 16), lambda i: (i, 0)),
            pl.BlockSpec((BN, 16), lambda i: (i, 0)),
            pl.BlockSpec((128, 128), lambda i: (0, 0)),
            pl.BlockSpec((1, 128), lambda i: (0, 0)),
            pl.BlockSpec((128, 128), lambda i: (0, 0)),
            pl.BlockSpec((1, 128), lambda i: (0, 0)),
            pl.BlockSpec((128, 16), lambda i: (0, 0)),
        ],
        out_specs=[pl.BlockSpec((BN, 16), lambda i: (i, 0))] * 4,
        out_shape=[jax.ShapeDtypeStruct((N, 16), f32)] * 4,
    )(x, cnt0, cnt1, nm_W1, nm_b1.reshape(1, 128), nm_W2,
      nm_b2.reshape(1, 128), g1_W)
    return outs


def _tc_edge(ea_p, em_W1, em_b1, em_W2, em_b2, nn_W1, nn_b1):
    BE = 2048
    grid = EP // BE

    def body(ea_ref, w1_ref, b1_ref, w2_ref, b2_ref, nw1_ref, nb1_ref, a_ref):
        ea = ea_ref[...]
        hh = jax.nn.relu(
            jnp.dot(ea, w1_ref[...], preferred_element_type=f32) + b1_ref[...])
        ee = jnp.dot(hh, w2_ref[...], preferred_element_type=f32) + b2_ref[...]
        a_ref[...] = jax.nn.relu(
            jnp.dot(ee, nw1_ref[...], preferred_element_type=f32) + nb1_ref[...])

    return pl.pallas_call(
        body,
        grid=(grid,),
        in_specs=[
            pl.BlockSpec((BE, 16), lambda i: (i, 0)),
            pl.BlockSpec((16, 128), lambda i: (0, 0)),
            pl.BlockSpec((1, 128), lambda i: (0, 0)),
            pl.BlockSpec((128, 16), lambda i: (0, 0)),
            pl.BlockSpec((1, 16), lambda i: (0, 0)),
            pl.BlockSpec((16, 32), lambda i: (0, 0)),
            pl.BlockSpec((1, 32), lambda i: (0, 0)),
        ],
        out_specs=pl.BlockSpec((BE, 32), lambda i: (i, 0)),
        out_shape=jax.ShapeDtypeStruct((EP, 32), f32),
    )(ea_p, em_W1, em_b1.reshape(1, 128), em_W2, em_b2.reshape(1, 16),
      nn_W1, nn_b1.reshape(1, 32))


def _tc_x1(acc0, acc1, dinv, xw, g1_b, root_W, root_b):
    BN = 1000

    def body(a0_ref, a1_ref, dinv_ref, xw_ref, b_ref, rw_ref, rb_ref,
             x1_ref, x1r_ref):
        dinv = dinv_ref[...]
        x1 = dinv * (a0_ref[...] + a1_ref[...]) \
            + 2.0 * dinv * dinv * xw_ref[...] + b_ref[...]
        x1_ref[...] = x1
        x1r_ref[...] = jnp.dot(x1, rw_ref[...],
                               preferred_element_type=f32) + rb_ref[...]

    return pl.pallas_call(
        body,
        grid=(N // BN,),
        in_specs=[
            pl.BlockSpec((BN, 16), lambda i: (i, 0)),
            pl.BlockSpec((BN, 16), lambda i: (i, 0)),
            pl.BlockSpec((BN, 16), lambda i: (i, 0)),
            pl.BlockSpec((BN, 16), lambda i: (i, 0)),
            pl.BlockSpec((1, 16), lambda i: (0, 0)),
            pl.BlockSpec((16, 16), lambda i: (0, 0)),
            pl.BlockSpec((1, 16), lambda i: (0, 0)),
        ],
        out_specs=[pl.BlockSpec((BN, 16), lambda i: (i, 0))] * 2,
        out_shape=[jax.ShapeDtypeStruct((N, 16), f32)] * 2,
    )(acc0, acc1, dinv, xw, g1_b.reshape(1, 16), root_W, root_b.reshape(1, 16))


def _tc_msg(x1g, a, W2r, B2r):
    BE = 2048
    grid = EP // BE

    def body(xg_ref, a_ref, w_ref, b_ref, msg_ref):
        xg = xg_ref[...]
        av = a_ref[...]
        q = (xg[:, :, None] * av[:, None, :]).reshape(BE, 512)
        msg_ref[...] = (
            jnp.dot(q, w_ref[...], preferred_element_type=f32)
            + jnp.dot(xg, b_ref[...], preferred_element_type=f32))

    return pl.pallas_call(
        body,
        grid=(grid,),
        in_specs=[
            pl.BlockSpec((BE, 16), lambda i: (i, 0)),
            pl.BlockSpec((BE, 32), lambda i: (i, 0)),
            pl.BlockSpec((512, 16), lambda i: (0, 0)),
            pl.BlockSpec((16, 16), lambda i: (0, 0)),
        ],
        out_specs=pl.BlockSpec((BE, 16), lambda i: (i, 0)),
        out_shape=jax.ShapeDtypeStruct((EP, 16), f32),
    )(x1g, a, W2r, B2r)


def _tc_x2(s0, s1, cnt, x1root, dinv, g2_W):
    BN = 1000

    def body(s0_ref, s1_ref, cnt_ref, x1r_ref, dinv_ref, g2_ref,
             x2_ref, u_ref, c0_ref):
        s = s0_ref[...] + s1_ref[...]
        x2 = jnp.tanh(s / jnp.maximum(cnt_ref[...], 1.0) + x1r_ref[...])
        xw2 = jnp.dot(x2, g2_ref[...], preferred_element_type=f32)
        dinv = dinv_ref[...]
        x2_ref[...] = x2
        u_ref[...] = dinv * xw2
        c0_ref[...] = 2.0 * dinv * xw2

    return pl.pallas_call(
        body,
        grid=(N // BN,),
        in_specs=[
            pl.BlockSpec((BN, 16), lambda i: (i, 0)),
            pl.BlockSpec((BN, 16), lambda i: (i, 0)),
            pl.BlockSpec((BN, 16), lambda i: (i, 0)),
            pl.BlockSpec((BN, 16), lambda i: (i, 0)),
            pl.BlockSpec((BN, 16), lambda i: (i, 0)),
            pl.BlockSpec((16, 16), lambda i: (0, 0)),
        ],
        out_specs=[pl.BlockSpec((BN, 16), lambda i: (i, 0))] * 3,
        out_shape=[jax.ShapeDtypeStruct((N, 16), f32)] * 3,
    )(s0, s1, cnt, x1root, dinv, g2_W)


def _tc_x3(maxp, c0, dinv, g2_b):
    def body(mp_ref, c0_ref, dinv_ref, b_ref, out_ref):
        s = pl.program_id(1)
        cur = mp_ref[0, 0, pl.ds(0, HALF), :]

        @pl.when(s == 0)
        def _():
            out_ref[0] = cur

        @pl.when(s != 0)
        def _():
            out_ref[0] = jnp.maximum(out_ref[0], cur)

        @pl.when(s == NS - 1)
        def _():
            out_ref[0] = (dinv_ref[0] * jnp.maximum(out_ref[0], c0_ref[0])
                          + b_ref[0])

    out = pl.pallas_call(
        body,
        grid=(NC, NS),
        in_specs=[
            pl.BlockSpec((1, 1, HP, 16), lambda c, s: (c, s, 0, 0)),
            pl.BlockSpec((1, HALF, 16), lambda c, s: (c, 0, 0)),
            pl.BlockSpec((1, HALF, 16), lambda c, s: (c, 0, 0)),
            pl.BlockSpec((1, 1, 16), lambda c, s: (0, 0, 0)),
        ],
        out_specs=pl.BlockSpec((1, HALF, 16), lambda c, s: (c, 0, 0)),
        out_shape=jax.ShapeDtypeStruct((NC, HALF, 16), f32),
    )(maxp, c0.reshape(NC, HALF, 16), dinv.reshape(NC, HALF, 16),
      g2_b.reshape(1, 1, 16))
    return out.reshape(N, 16)


def _tc_final(x1, x2, x3, batch1, scales, orientation,
              gate_W1, gate_b1, gate_W2, gate_b2,
              m_W1, m_b1, m_W2, m_b2, m_W3, m_b3):
    def body(x1_ref, x2_ref, x3_ref, b_ref, sc_ref, or_ref,
             gw1_ref, gb1_ref, gw2_ref, gb2_ref,
             mw1_ref, mb1_ref, mw2_ref, mb2_ref, mw3_ref, mb3_ref, out_ref):
        xc = jnp.concatenate([x1_ref[...], x2_ref[...], x3_ref[...]], axis=1)
        g1 = jax.nn.relu(
            jnp.dot(xc, gw1_ref[...], preferred_element_type=f32) + gb1_ref[...])
        gate = jnp.dot(g1, gw2_ref[...], preferred_element_type=f32) + gb2_ref[...]
        b = b_ref[...]
        gid = jax.lax.broadcasted_iota(i32, (1, 8), 1)
        mask = (b == gid)
        gm = jnp.where(mask, gate, -3e38)
        gmax = jnp.max(gm, axis=0, keepdims=True)
        ev = jnp.where(mask, jnp.exp(gate - gmax), 0.0)
        den = jnp.sum(ev, axis=0, keepdims=True)
        w = ev / (den + 1e-16)
        pooled = lax.dot_general(w, xc, (((0,), (0,)), ((), ())),
                                 preferred_element_type=f32)
        feats = jnp.concatenate(
            [pooled, jnp.log(sc_ref[...] + 1e-5), or_ref[...]], axis=1)
        o = jax.nn.relu(
            jnp.dot(feats, mw1_ref[...], preferred_element_type=f32) + mb1_ref[...])
        o = jax.nn.relu(
            jnp.dot(o, mw2_ref[...], preferred_element_type=f32) + mb2_ref[...])
        out_ref[...] = jnp.dot(o, mw3_ref[...],
                               preferred_element_type=f32) + mb3_ref[...]

    return pl.pallas_call(
        body,
        out_shape=jax.ShapeDtypeStruct((8, 8), f32),
    )(x1, x2, x3, batch1, scales, orientation,
      gate_W1, gate_b1.reshape(1, 256), gate_W2, gate_b2.reshape(1, 1),
      m_W1, m_b1.reshape(1, 16), m_W2, m_b2.reshape(1, 8),
      m_W3, m_b3.reshape(1, 8))


def kernel(x, edge_index, edge_attr, batch, scales, orientation,
           nm_W1, nm_b1, nm_W2, nm_b2,
           em_W1, em_b1, em_W2, em_b2,
           g1_W, g1_b,
           nn_W1, nn_b1, nn_W2, nn_b2,
           root_W, root_b,
           g2_W, g2_b,
           gate_W1, gate_b1, gate_W2, gate_b2,
           m_W1, m_b1, m_W2, m_b2, m_W3, m_b3):
    src = edge_index[0]
    dst = edge_index[1]
    pad = EP - E
    src_p = jnp.concatenate([src, jnp.zeros((pad,), i32)])
    dst_p = jnp.concatenate([dst, jnp.full((pad,), N, i32)])
    ea_p = jnp.concatenate([edge_attr, jnp.zeros((pad, 16), f32)], axis=0)
    ones_rows = jnp.ones((CH, 16), f32)
    zeros_acc = jnp.zeros((NP, 16), f32)
    neg_rows = jnp.full((HP, 16), -3e38, f32)

    cntp = _sc_count(dst_p, ones_rows, zeros_acc)[:, :N]
    y, xw, dinv, cnt = _tc_prep(x, cntp[0], cntp[1],
                                nm_W1, nm_b1, nm_W2, nm_b2, g1_W)
    accp = _sc_gather_scatter(src_p, dst_p, y, zeros_acc)[:, :N]
    a = _tc_edge(ea_p, em_W1, em_b1, em_W2, em_b2, nn_W1, nn_b1)
    x1, x1root = _tc_x1(accp[0], accp[1], dinv, xw, g1_b, root_W, root_b)
    x1g = _sc_gather_rows(src_p, x1)
    W2r = nn_W2.reshape(32, 16, 16).transpose(1, 0, 2).reshape(512, 16)
    B2r = nn_b2.reshape(16, 16)
    msg = _tc_msg(x1g, a, W2r, B2r)
    sp = _sc_scatter_rows(dst_p, msg, zeros_acc)[:, :N]
    x2, u, c0 = _tc_x2(sp[0], sp[1], cnt, x1root, dinv, g2_W)
    maxp = _sc_scatter_max(src_p, dst_p, u, neg_rows)
    x3 = _tc_x3(maxp, c0, dinv, g2_b)
    return _tc_final(x1, x2, x3, batch.reshape(N, 1).astype(i32),
                     scales, orientation,
                     gate_W1, gate_b1, gate_W2, gate_b2,
                     m_W1, m_b1, m_W2, m_b2, m_W3, m_b3)


# speedup vs baseline: 4.5002x; 1.1284x over previous
import functools

import jax
import jax.numpy as jnp
from jax import lax
from jax.experimental import pallas as pl
from jax.experimental.pallas import tpu as pltpu
from jax.experimental.pallas import tpu_sc as plsc

f32 = jnp.float32
i32 = jnp.int32

N = 10000
E = 160000
EP = 163840
CH = 128
NC = 2
NS = 16
NW = NC * NS
NP = 10240
HALF = N // 2
HP = HALF + 8

_mesh = lambda: plsc.VectorSubcoreMesh(core_axis_name="c", subcore_axis_name="s")


G4 = 4

def _sc_count(dst2, ones_rows, zeros_acc):
    @functools.partial(
        pl.kernel,
        out_type=jax.ShapeDtypeStruct((NC, NP, 16), f32),
        mesh=_mesh(),
        compiler_params=pltpu.CompilerParams(use_tc_tiling_on_sc=False),
        scratch_types=[
            pltpu.VMEM((EP // NW // CH, CH), i32),
            pltpu.VMEM((CH, 16), f32),
            pltpu.VMEM_SHARED((NC, NP, 16), f32),
            pltpu.SemaphoreType.DMA,
        ],
    )
    def k(dst_hbm, ones_hbm, zeros_hbm, out_hbm, didx, ones_v, acc, ssem):
        c = lax.axis_index("c")
        s = lax.axis_index("s")
        r0 = NP // NS
        pltpu.sync_copy(zeros_hbm.at[pl.ds(s * r0, r0)],
                        acc.at[c].at[pl.ds(s * r0, r0)])
        pltpu.sync_copy(ones_hbm, ones_v)
        wid = c * NS + s
        nch = EP // NW // CH
        pltpu.sync_copy(dst_hbm.at[pl.ds(wid * nch, nch)], didx)
        plsc.subcore_barrier()

        def grp(gi, carry):
            ds_ = [pltpu.async_copy(ones_v, acc.at[c].at[didx.at[gi * G4 + j]],
                                    ssem, add=True) for j in range(G4)]
            for d in ds_:
                d.wait()
            return carry

        lax.fori_loop(0, nch // G4, grp, 0)
        plsc.subcore_barrier()
        pltpu.sync_copy(acc.at[c].at[pl.ds(s * r0, r0)],
                        out_hbm.at[c].at[pl.ds(s * r0, r0)])

    return k(dst2, ones_rows, zeros_acc)


def _sc_gather_scatter(src2, dst2, table, zeros_acc):
    @functools.partial(
        pl.kernel,
        out_type=jax.ShapeDtypeStruct((NC, NP, 16), f32),
        mesh=_mesh(),
        compiler_params=pltpu.CompilerParams(use_tc_tiling_on_sc=False),
        scratch_types=[
            pltpu.VMEM((EP // NW // CH, CH), i32),
            pltpu.VMEM((EP // NW // CH, CH), i32),
            pltpu.VMEM((G4 * CH, 16), f32),
            pltpu.VMEM_SHARED((NC, NP, 16), f32),
            pltpu.SemaphoreType.DMA,
            pltpu.SemaphoreType.DMA,
        ],
    )
    def k(src_hbm, dst_hbm, tab_hbm, zeros_hbm, out_hbm,
          sidx, didx, rows_v, acc, gsem, ssem):
        c = lax.axis_index("c")
        s = lax.axis_index("s")
        r0 = NP // NS
        pltpu.sync_copy(zeros_hbm.at[pl.ds(s * r0, r0)],
                        acc.at[c].at[pl.ds(s * r0, r0)])
        wid = c * NS + s
        nch = EP // NW // CH
        pltpu.sync_copy(src_hbm.at[pl.ds(wid * nch, nch)], sidx)
        pltpu.sync_copy(dst_hbm.at[pl.ds(wid * nch, nch)], didx)
        plsc.subcore_barrier()

        def grp(gi, carry):
            gs = [pltpu.async_copy(tab_hbm.at[sidx.at[gi * G4 + j]],
                                   rows_v.at[pl.ds(j * CH, CH)], gsem)
                  for j in range(G4)]
            for d in gs:
                d.wait()
            ss = [pltpu.async_copy(rows_v.at[pl.ds(j * CH, CH)],
                                   acc.at[c].at[didx.at[gi * G4 + j]],
                                   ssem, add=True) for j in range(G4)]
            for d in ss:
                d.wait()
            return carry

        lax.fori_loop(0, nch // G4, grp, 0)
        plsc.subcore_barrier()
        pltpu.sync_copy(acc.at[c].at[pl.ds(s * r0, r0)],
                        out_hbm.at[c].at[pl.ds(s * r0, r0)])

    return k(src2, dst2, table, zeros_acc)


def _sc_scatter_rows(dst2, rows, zeros_acc):
    @functools.partial(
        pl.kernel,
        out_type=jax.ShapeDtypeStruct((NC, NP, 16), f32),
        mesh=_mesh(),
        compiler_params=pltpu.CompilerParams(use_tc_tiling_on_sc=False),
        scratch_types=[
            pltpu.VMEM((EP // NW // CH, CH), i32),
            pltpu.VMEM((G4 * CH, 16), f32),
            pltpu.VMEM_SHARED((NC, NP, 16), f32),
            pltpu.SemaphoreType.DMA,
        ],
    )
    def k(dst_hbm, rows_hbm, zeros_hbm, out_hbm, didx, rows_v, acc, ssem):
        c = lax.axis_index("c")
        s = lax.axis_index("s")
        r0 = NP // NS
        pltpu.sync_copy(zeros_hbm.at[pl.ds(s * r0, r0)],
                        acc.at[c].at[pl.ds(s * r0, r0)])
        wid = c * NS + s
        nch = EP // NW // CH
        pltpu.sync_copy(dst_hbm.at[pl.ds(wid * nch, nch)], didx)
        plsc.subcore_barrier()
        ebase = wid * (EP // NW)

        def grp(gi, carry):
            pltpu.sync_copy(rows_hbm.at[pl.ds(ebase + gi * G4 * CH, G4 * CH)],
                            rows_v)
            ss = [pltpu.async_copy(rows_v.at[pl.ds(j * CH, CH)],
                                   acc.at[c].at[didx.at[gi * G4 + j]],
                                   ssem, add=True) for j in range(G4)]
            for d in ss:
                d.wait()
            return carry

        lax.fori_loop(0, (EP // NW) // (G4 * CH), grp, 0)
        plsc.subcore_barrier()
        pltpu.sync_copy(acc.at[c].at[pl.ds(s * r0, r0)],
                        out_hbm.at[c].at[pl.ds(s * r0, r0)])

    return k(dst2, rows, zeros_acc)


def _sc_gather_rows(src2, table):
    @functools.partial(
        pl.kernel,
        out_type=jax.ShapeDtypeStruct((EP, 16), f32),
        mesh=_mesh(),
        compiler_params=pltpu.CompilerParams(use_tc_tiling_on_sc=False),
        scratch_types=[
            pltpu.VMEM((EP // NW // CH, CH), i32),
            pltpu.VMEM((G4 * CH, 16), f32),
            pltpu.SemaphoreType.DMA,
        ],
    )
    def k(src_hbm, tab_hbm, out_hbm, sidx, rows_v, gsem):
        c = lax.axis_index("c")
        s = lax.axis_index("s")
        wid = c * NS + s
        nch = EP // NW // CH
        pltpu.sync_copy(src_hbm.at[pl.ds(wid * nch, nch)], sidx)
        ebase = wid * (EP // NW)

        def grp(gi, carry):
            gs = [pltpu.async_copy(tab_hbm.at[sidx.at[gi * G4 + j]],
                                   rows_v.at[pl.ds(j * CH, CH)], gsem)
                  for j in range(G4)]
            for d in gs:
                d.wait()
            pltpu.sync_copy(rows_v,
                            out_hbm.at[pl.ds(ebase + gi * G4 * CH, G4 * CH)])
            return carry

        lax.fori_loop(0, (EP // NW) // (G4 * CH), grp, 0)

    return k(src2, table)


def _sc_scatter_max(src2, dst2, table, neg_rows):
    @functools.partial(
        pl.kernel,
        out_type=jax.ShapeDtypeStruct((NC, NS, HP, 16), f32),
        mesh=_mesh(),
        compiler_params=pltpu.CompilerParams(use_tc_tiling_on_sc=False),
        scratch_types=[
            pltpu.VMEM((EP // NS // CH, CH), i32),
            pltpu.VMEM((EP // NS // CH, CH), i32),
            pltpu.VMEM((G4 * CH, 16), f32),
            pltpu.VMEM((HP, 16), f32),
            pltpu.SemaphoreType.DMA,
        ],
    )
    def k(src_hbm, dst_hbm, tab_hbm, neg_hbm, out_hbm,
          sidx, didx, rows_v, acc_v, gsem):
        c = lax.axis_index("c")
        s = lax.axis_index("s")
        pltpu.sync_copy(neg_hbm, acc_v)
        nch = EP // NS // CH
        pltpu.sync_copy(src_hbm.at[pl.ds(s * nch, nch)], sidx)
        pltpu.sync_copy(dst_hbm.at[pl.ds(s * nch, nch)], didx)
        nbase = c * HALF

        def grp(gi, carry):
            gs = [pltpu.async_copy(tab_hbm.at[sidx.at[gi * G4 + j]],
                                   rows_v.at[pl.ds(j * CH, CH)], gsem)
                  for j in range(G4)]
            for d in gs:
                d.wait()
            for j in range(G4):
                for g in range(CH // 16):
                    dv = didx[gi * G4 + j, pl.ds(g * 16, 16)]
                    dl = dv - nbase
                    ok = (dl >= 0) & (dl < HALF)
                    idx16 = jnp.where(ok, dl, HALF)
                    for l in range(16):
                        ri = idx16[l]
                        row = rows_v[j * CH + g * 16 + l, :]
                        acc_v[ri, :] = jnp.maximum(acc_v[ri, :], row)
            return carry

        lax.fori_loop(0, (EP // NS) // (G4 * CH), grp, 0)
        pltpu.sync_copy(acc_v, out_hbm.at[c].at[s])

    return k(src2, dst2, table, neg_rows)


def _tc_prep(x, cnt0, cnt1, nm_W1, nm_b1, nm_W2, nm_b2, g1_W):
    BN = 1000
    grid = N // BN

    def body(x_ref, c0_ref, c1_ref, w1_ref, b1_ref, w2_ref, b2_ref, g1_ref,
             y_ref, xw_ref, dinv_ref, cnt_ref):
        xb = x_ref[...]
        h = jax.nn.relu(
            jnp.dot(xb, w1_ref[...], preferred_element_type=f32) + b1_ref[...])
        h = jnp.dot(h, w2_ref[...], preferred_element_type=f32) + b2_ref[...]
        xw = jnp.dot(h, g1_ref[...], preferred_element_type=f32)
        cnt = c0_ref[...] + c1_ref[...]
        dinv = lax.rsqrt(cnt + 2.0)
        y_ref[...] = dinv * xw
        xw_ref[...] = xw
        dinv_ref[...] = dinv
        cnt_ref[...] = cnt

    outs = pl.pallas_call(
        body,
        grid=(grid,),
        in_specs=[
            pl.BlockSpec((BN, 128), lambda i: (i, 0)),
            pl.BlockSpec((BN, 16), lambda i: (i, 0)),
            pl.BlockSpec((BN, 16), lambda i: (i, 0)),
            pl.BlockSpec((128, 128), lambda i: (0, 0)),
            pl.BlockSpec((1, 128), lambda i: (0, 0)),
            pl.BlockSpec((128, 128), lambda i: (0, 0)),
            pl.BlockSpec((1, 128), lambda i: (0, 0)),
            pl.BlockSpec((128, 16), lambda i: (0, 0)),
        ],
        out_specs=[pl.BlockSpec((BN, 16), lambda i: (i, 0))] * 4,
        out_shape=[jax.ShapeDtypeStruct((N, 16), f32)] * 4,
    )(x, cnt0, cnt1, nm_W1, nm_b1.reshape(1, 128), nm_W2,
      nm_b2.reshape(1, 128), g1_W)
    return outs


def _tc_edge(ea_p, em_W1, em_b1, em_W2, em_b2, nn_W1, nn_b1):
    BE = 2048
    grid = EP // BE

    def body(ea_ref, w1_ref, b1_ref, w2_ref, b2_ref, nw1_ref, nb1_ref, a_ref):
        ea = ea_ref[...]
        hh = jax.nn.relu(
            jnp.dot(ea, w1_ref[...], preferred_element_type=f32) + b1_ref[...])
        ee = jnp.dot(hh, w2_ref[...], preferred_element_type=f32) + b2_ref[...]
        a_ref[...] = jax.nn.relu(
            jnp.dot(ee, nw1_ref[...], preferred_element_type=f32) + nb1_ref[...])

    return pl.pallas_call(
        body,
        grid=(grid,),
        in_specs=[
            pl.BlockSpec((BE, 16), lambda i: (i, 0)),
            pl.BlockSpec((16, 128), lambda i: (0, 0)),
            pl.BlockSpec((1, 128), lambda i: (0, 0)),
            pl.BlockSpec((128, 16), lambda i: (0, 0)),
            pl.BlockSpec((1, 16), lambda i: (0, 0)),
            pl.BlockSpec((16, 32), lambda i: (0, 0)),
            pl.BlockSpec((1, 32), lambda i: (0, 0)),
        ],
        out_specs=pl.BlockSpec((BE, 32), lambda i: (i, 0)),
        out_shape=jax.ShapeDtypeStruct((EP, 32), f32),
    )(ea_p, em_W1, em_b1.reshape(1, 128), em_W2, em_b2.reshape(1, 16),
      nn_W1, nn_b1.reshape(1, 32))


def _tc_x1(acc0, acc1, dinv, xw, g1_b, root_W, root_b):
    BN = 1000

    def body(a0_ref, a1_ref, dinv_ref, xw_ref, b_ref, rw_ref, rb_ref,
             x1_ref, x1r_ref):
        dinv = dinv_ref[...]
        x1 = dinv * (a0_ref[...] + a1_ref[...]) \
            + 2.0 * dinv * dinv * xw_ref[...] + b_ref[...]
        x1_ref[...] = x1
        x1r_ref[...] = jnp.dot(x1, rw_ref[...],
                               preferred_element_type=f32) + rb_ref[...]

    return pl.pallas_call(
        body,
        grid=(N // BN,),
        in_specs=[
            pl.BlockSpec((BN, 16), lambda i: (i, 0)),
            pl.BlockSpec((BN, 16), lambda i: (i, 0)),
            pl.BlockSpec((BN, 16), lambda i: (i, 0)),
            pl.BlockSpec((BN, 16), lambda i: (i, 0)),
            pl.BlockSpec((1, 16), lambda i: (0, 0)),
            pl.BlockSpec((16, 16), lambda i: (0, 0)),
            pl.BlockSpec((1, 16), lambda i: (0, 0)),
        ],
        out_specs=[pl.BlockSpec((BN, 16), lambda i: (i, 0))] * 2,
        out_shape=[jax.ShapeDtypeStruct((N, 16), f32)] * 2,
    )(acc0, acc1, dinv, xw, g1_b.reshape(1, 16), root_W, root_b.reshape(1, 16))


def _tc_msg(x1g, a, W2r, B2r):
    BE = 2048
    grid = EP // BE

    def body(xg_ref, a_ref, w_ref, b_ref, msg_ref):
        xg = xg_ref[...]
        av = a_ref[...]
        q = (xg[:, :, None] * av[:, None, :]).reshape(BE, 512)
        msg_ref[...] = (
            jnp.dot(q, w_ref[...], preferred_element_type=f32)
            + jnp.dot(xg, b_ref[...], preferred_element_type=f32))

    return pl.pallas_call(
        body,
        grid=(grid,),
        in_specs=[
            pl.BlockSpec((BE, 16), lambda i: (i, 0)),
            pl.BlockSpec((BE, 32), lambda i: (i, 0)),
            pl.BlockSpec((512, 16), lambda i: (0, 0)),
            pl.BlockSpec((16, 16), lambda i: (0, 0)),
        ],
        out_specs=pl.BlockSpec((BE, 16), lambda i: (i, 0)),
        out_shape=jax.ShapeDtypeStruct((EP, 16), f32),
    )(x1g, a, W2r, B2r)


def _tc_x2(s0, s1, cnt, x1root, dinv, g2_W):
    BN = 1000

    def body(s0_ref, s1_ref, cnt_ref, x1r_ref, dinv_ref, g2_ref,
             x2_ref, u_ref, c0_ref):
        s = s0_ref[...] + s1_ref[...]
        x2 = jnp.tanh(s / jnp.maximum(cnt_ref[...], 1.0) + x1r_ref[...])
        xw2 = jnp.dot(x2, g2_ref[...], preferred_element_type=f32)
        dinv = dinv_ref[...]
        x2_ref[...] = x2
        u_ref[...] = dinv * xw2
        c0_ref[...] = 2.0 * dinv * xw2

    return pl.pallas_call(
        body,
        grid=(N // BN,),
        in_specs=[
            pl.BlockSpec((BN, 16), lambda i: (i, 0)),
            pl.BlockSpec((BN, 16), lambda i: (i, 0)),
            pl.BlockSpec((BN, 16), lambda i: (i, 0)),
            pl.BlockSpec((BN, 16), lambda i: (i, 0)),
            pl.BlockSpec((BN, 16), lambda i: (i, 0)),
            pl.BlockSpec((16, 16), lambda i: (0, 0)),
        ],
        out_specs=[pl.BlockSpec((BN, 16), lambda i: (i, 0))] * 3,
        out_shape=[jax.ShapeDtypeStruct((N, 16), f32)] * 3,
    )(s0, s1, cnt, x1root, dinv, g2_W)


def _tc_x3(maxp, c0, dinv, g2_b):
    def body(mp_ref, c0_ref, dinv_ref, b_ref, out_ref):
        s = pl.program_id(1)
        cur = mp_ref[0, 0, pl.ds(0, HALF), :]

        @pl.when(s == 0)
        def _():
            out_ref[0] = cur

        @pl.when(s != 0)
        def _():
            out_ref[0] = jnp.maximum(out_ref[0], cur)

        @pl.when(s == NS - 1)
        def _():
            out_ref[0] = (dinv_ref[0] * jnp.maximum(out_ref[0], c0_ref[0])
                          + b_ref[0])

    out = pl.pallas_call(
        body,
        grid=(NC, NS),
        in_specs=[
            pl.BlockSpec((1, 1, HP, 16), lambda c, s: (c, s, 0, 0)),
            pl.BlockSpec((1, HALF, 16), lambda c, s: (c, 0, 0)),
            pl.BlockSpec((1, HALF, 16), lambda c, s: (c, 0, 0)),
            pl.BlockSpec((1, 1, 16), lambda c, s: (0, 0, 0)),
        ],
        out_specs=pl.BlockSpec((1, HALF, 16), lambda c, s: (c, 0, 0)),
        out_shape=jax.ShapeDtypeStruct((NC, HALF, 16), f32),
    )(maxp, c0.reshape(NC, HALF, 16), dinv.reshape(NC, HALF, 16),
      g2_b.reshape(1, 1, 16))
    return out.reshape(N, 16)


def _tc_final(x1, x2, x3, batch1, scales, orientation,
              gate_W1, gate_b1, gate_W2, gate_b2,
              m_W1, m_b1, m_W2, m_b2, m_W3, m_b3):
    def body(x1_ref, x2_ref, x3_ref, b_ref, sc_ref, or_ref,
             gw1_ref, gb1_ref, gw2_ref, gb2_ref,
             mw1_ref, mb1_ref, mw2_ref, mb2_ref, mw3_ref, mb3_ref, out_ref):
        xc = jnp.concatenate([x1_ref[...], x2_ref[...], x3_ref[...]], axis=1)
        g1 = jax.nn.relu(
            jnp.dot(xc, gw1_ref[...], preferred_element_type=f32) + gb1_ref[...])
        gate = jnp.dot(g1, gw2_ref[...], preferred_element_type=f32) + gb2_ref[...]
        b = b_ref[...]
        gid = jax.lax.broadcasted_iota(i32, (1, 8), 1)
        mask = (b == gid)
        gm = jnp.where(mask, gate, -3e38)
        gmax = jnp.max(gm, axis=0, keepdims=True)
        ev = jnp.where(mask, jnp.exp(gate - gmax), 0.0)
        den = jnp.sum(ev, axis=0, keepdims=True)
        w = ev / (den + 1e-16)
        pooled = lax.dot_general(w, xc, (((0,), (0,)), ((), ())),
                                 preferred_element_type=f32)
        feats = jnp.concatenate(
            [pooled, jnp.log(sc_ref[...] + 1e-5), or_ref[...]], axis=1)
        o = jax.nn.relu(
            jnp.dot(feats, mw1_ref[...], preferred_element_type=f32) + mb1_ref[...])
        o = jax.nn.relu(
            jnp.dot(o, mw2_ref[...], preferred_element_type=f32) + mb2_ref[...])
        out_ref[...] = jnp.dot(o, mw3_ref[...],
                               preferred_element_type=f32) + mb3_ref[...]

    return pl.pallas_call(
        body,
        out_shape=jax.ShapeDtypeStruct((8, 8), f32),
    )(x1, x2, x3, batch1, scales, orientation,
      gate_W1, gate_b1.reshape(1, 256), gate_W2, gate_b2.reshape(1, 1),
      m_W1, m_b1.reshape(1, 16), m_W2, m_b2.reshape(1, 8),
      m_W3, m_b3.reshape(1, 8))


def kernel(x, edge_index, edge_attr, batch, scales, orientation,
           nm_W1, nm_b1, nm_W2, nm_b2,
           em_W1, em_b1, em_W2, em_b2,
           g1_W, g1_b,
           nn_W1, nn_b1, nn_W2, nn_b2,
           root_W, root_b,
           g2_W, g2_b,
           gate_W1, gate_b1, gate_W2, gate_b2,
           m_W1, m_b1, m_W2, m_b2, m_W3, m_b3):
    src = edge_index[0]
    dst = edge_index[1]
    pad = EP - E
    src2 = jnp.concatenate([src, jnp.zeros((pad,), i32)]).reshape(EP // CH, CH)
    dst2 = jnp.concatenate([dst, jnp.full((pad,), N, i32)]).reshape(EP // CH, CH)
    ea_p = jnp.concatenate([edge_attr, jnp.zeros((pad, 16), f32)], axis=0)
    ones_rows = jnp.ones((CH, 16), f32)
    zeros_acc = jnp.zeros((NP, 16), f32)
    neg_rows = jnp.full((HP, 16), -3e38, f32)

    cntp = _sc_count(dst2, ones_rows, zeros_acc)[:, :N]
    y, xw, dinv, cnt = _tc_prep(x, cntp[0], cntp[1],
                                nm_W1, nm_b1, nm_W2, nm_b2, g1_W)
    accp = _sc_gather_scatter(src2, dst2, y, zeros_acc)[:, :N]
    a = _tc_edge(ea_p, em_W1, em_b1, em_W2, em_b2, nn_W1, nn_b1)
    x1, x1root = _tc_x1(accp[0], accp[1], dinv, xw, g1_b, root_W, root_b)
    x1g = _sc_gather_rows(src2, x1)
    W2r = nn_W2.reshape(32, 16, 16).transpose(1, 0, 2).reshape(512, 16)
    B2r = nn_b2.reshape(16, 16)
    msg = _tc_msg(x1g, a, W2r, B2r)
    sp = _sc_scatter_rows(dst2, msg, zeros_acc)[:, :N]
    x2, u, c0 = _tc_x2(sp[0], sp[1], cnt, x1root, dinv, g2_W)
    maxp = _sc_scatter_max(src2, dst2, u, neg_rows)
    x3 = _tc_x3(maxp, c0, dinv, g2_b)
    return _tc_final(x1, x2, x3, batch.reshape(N, 1).astype(i32),
                     scales, orientation,
                     gate_W1, gate_b1, gate_W2, gate_b2,
                     m_W1, m_b1, m_W2, m_b2, m_W3, m_b3)


# speedup vs baseline: 5.1248x; 1.1388x over previous
import functools

import jax
import jax.numpy as jnp
from jax import lax
from jax.experimental import pallas as pl
from jax.experimental.pallas import tpu as pltpu
from jax.experimental.pallas import tpu_sc as plsc

f32 = jnp.float32
i32 = jnp.int32

N = 10000
E = 160000
EP = 163840
CH = 128
NC = 2
NS = 16
NW = NC * NS
NP = 10240
HALF = N // 2
HP = 5120

_mesh = lambda: plsc.VectorSubcoreMesh(core_axis_name="c", subcore_axis_name="s")


G4 = 4

def _sc_count(dst2, ones_rows, zeros_acc):
    @functools.partial(
        pl.kernel,
        out_type=jax.ShapeDtypeStruct((NC, NP, 16), f32),
        mesh=_mesh(),
        compiler_params=pltpu.CompilerParams(use_tc_tiling_on_sc=False),
        scratch_types=[
            pltpu.VMEM((EP // NW // CH, CH), i32),
            pltpu.VMEM((CH, 16), f32),
            pltpu.VMEM_SHARED((NC, NP, 16), f32),
            pltpu.SemaphoreType.DMA,
        ],
    )
    def k(dst_hbm, ones_hbm, zeros_hbm, out_hbm, didx, ones_v, acc, ssem):
        c = lax.axis_index("c")
        s = lax.axis_index("s")
        r0 = NP // NS
        pltpu.sync_copy(zeros_hbm.at[pl.ds(s * r0, r0)],
                        acc.at[c].at[pl.ds(s * r0, r0)])
        pltpu.sync_copy(ones_hbm, ones_v)
        wid = c * NS + s
        nch = EP // NW // CH
        pltpu.sync_copy(dst_hbm.at[pl.ds(wid * nch, nch)], didx)
        plsc.subcore_barrier()

        def grp(gi, carry):
            ds_ = [pltpu.async_copy(ones_v, acc.at[c].at[didx.at[gi * G4 + j]],
                                    ssem, add=True) for j in range(G4)]
            for d in ds_:
                d.wait()
            return carry

        lax.fori_loop(0, nch // G4, grp, 0)
        plsc.subcore_barrier()
        pltpu.sync_copy(acc.at[c].at[pl.ds(s * r0, r0)],
                        out_hbm.at[c].at[pl.ds(s * r0, r0)])

    return k(dst2, ones_rows, zeros_acc)


def _sc_gather_scatter(src2, dst2, table, zeros_acc):
    @functools.partial(
        pl.kernel,
        out_type=jax.ShapeDtypeStruct((NC, NP, 16), f32),
        mesh=_mesh(),
        compiler_params=pltpu.CompilerParams(use_tc_tiling_on_sc=False),
        scratch_types=[
            pltpu.VMEM((EP // NW // CH, CH), i32),
            pltpu.VMEM((EP // NW // CH, CH), i32),
            pltpu.VMEM((G4 * CH, 16), f32),
            pltpu.VMEM_SHARED((NC, NP, 16), f32),
            pltpu.SemaphoreType.DMA,
            pltpu.SemaphoreType.DMA,
        ],
    )
    def k(src_hbm, dst_hbm, tab_hbm, zeros_hbm, out_hbm,
          sidx, didx, rows_v, acc, gsem, ssem):
        c = lax.axis_index("c")
        s = lax.axis_index("s")
        r0 = NP // NS
        pltpu.sync_copy(zeros_hbm.at[pl.ds(s * r0, r0)],
                        acc.at[c].at[pl.ds(s * r0, r0)])
        wid = c * NS + s
        nch = EP // NW // CH
        pltpu.sync_copy(src_hbm.at[pl.ds(wid * nch, nch)], sidx)
        pltpu.sync_copy(dst_hbm.at[pl.ds(wid * nch, nch)], didx)
        plsc.subcore_barrier()

        def grp(gi, carry):
            gs = [pltpu.async_copy(tab_hbm.at[sidx.at[gi * G4 + j]],
                                   rows_v.at[pl.ds(j * CH, CH)], gsem)
                  for j in range(G4)]
            for d in gs:
                d.wait()
            ss = [pltpu.async_copy(rows_v.at[pl.ds(j * CH, CH)],
                                   acc.at[c].at[didx.at[gi * G4 + j]],
                                   ssem, add=True) for j in range(G4)]
            for d in ss:
                d.wait()
            return carry

        lax.fori_loop(0, nch // G4, grp, 0)
        plsc.subcore_barrier()
        pltpu.sync_copy(acc.at[c].at[pl.ds(s * r0, r0)],
                        out_hbm.at[c].at[pl.ds(s * r0, r0)])

    return k(src2, dst2, table, zeros_acc)


def _sc_scatter_rows(dst2, rows, zeros_acc):
    @functools.partial(
        pl.kernel,
        out_type=jax.ShapeDtypeStruct((NC, NP, 16), f32),
        mesh=_mesh(),
        compiler_params=pltpu.CompilerParams(use_tc_tiling_on_sc=False),
        scratch_types=[
            pltpu.VMEM((EP // NW // CH, CH), i32),
            pltpu.VMEM((G4 * CH, 16), f32),
            pltpu.VMEM_SHARED((NC, NP, 16), f32),
            pltpu.SemaphoreType.DMA,
        ],
    )
    def k(dst_hbm, rows_hbm, zeros_hbm, out_hbm, didx, rows_v, acc, ssem):
        c = lax.axis_index("c")
        s = lax.axis_index("s")
        r0 = NP // NS
        pltpu.sync_copy(zeros_hbm.at[pl.ds(s * r0, r0)],
                        acc.at[c].at[pl.ds(s * r0, r0)])
        wid = c * NS + s
        nch = EP // NW // CH
        pltpu.sync_copy(dst_hbm.at[pl.ds(wid * nch, nch)], didx)
        plsc.subcore_barrier()
        ebase = wid * (EP // NW)

        def grp(gi, carry):
            pltpu.sync_copy(rows_hbm.at[pl.ds(ebase + gi * G4 * CH, G4 * CH)],
                            rows_v)
            ss = [pltpu.async_copy(rows_v.at[pl.ds(j * CH, CH)],
                                   acc.at[c].at[didx.at[gi * G4 + j]],
                                   ssem, add=True) for j in range(G4)]
            for d in ss:
                d.wait()
            return carry

        lax.fori_loop(0, (EP // NW) // (G4 * CH), grp, 0)
        plsc.subcore_barrier()
        pltpu.sync_copy(acc.at[c].at[pl.ds(s * r0, r0)],
                        out_hbm.at[c].at[pl.ds(s * r0, r0)])

    return k(dst2, rows, zeros_acc)


def _sc_gather_rows(src2, table):
    @functools.partial(
        pl.kernel,
        out_type=jax.ShapeDtypeStruct((EP, 16), f32),
        mesh=_mesh(),
        compiler_params=pltpu.CompilerParams(use_tc_tiling_on_sc=False),
        scratch_types=[
            pltpu.VMEM((EP // NW // CH, CH), i32),
            pltpu.VMEM((G4 * CH, 16), f32),
            pltpu.SemaphoreType.DMA,
        ],
    )
    def k(src_hbm, tab_hbm, out_hbm, sidx, rows_v, gsem):
        c = lax.axis_index("c")
        s = lax.axis_index("s")
        wid = c * NS + s
        nch = EP // NW // CH
        pltpu.sync_copy(src_hbm.at[pl.ds(wid * nch, nch)], sidx)
        ebase = wid * (EP // NW)

        def grp(gi, carry):
            gs = [pltpu.async_copy(tab_hbm.at[sidx.at[gi * G4 + j]],
                                   rows_v.at[pl.ds(j * CH, CH)], gsem)
                  for j in range(G4)]
            for d in gs:
                d.wait()
            pltpu.sync_copy(rows_v,
                            out_hbm.at[pl.ds(ebase + gi * G4 * CH, G4 * CH)])
            return carry

        lax.fori_loop(0, (EP // NW) // (G4 * CH), grp, 0)

    return k(src2, table)


def _sc_scatter_max(src2, dst2, table, neg_rows):
    @functools.partial(
        pl.kernel,
        out_type=[jax.ShapeDtypeStruct((NC, HP, 16), f32),
                  jax.ShapeDtypeStruct((NC, NS, HP, 16), f32)],
        mesh=_mesh(),
        compiler_params=pltpu.CompilerParams(use_tc_tiling_on_sc=False),
        scratch_types=[
            pltpu.VMEM((40, CH), i32),
            pltpu.VMEM((40, CH), i32),
            pltpu.VMEM((G4 * CH, 16), f32),
            pltpu.VMEM((HP, 16), f32),
            pltpu.VMEM((NS, 64, 16), f32),
            pltpu.VMEM((64, 16), f32),
            pltpu.SemaphoreType.DMA,
        ],
    )
    def k(src_hbm, dst_hbm, tab_hbm, neg_hbm, out_hbm, part_hbm,
          sidx, didx, rows_v, acc_v, stage, outbuf, gsem):
        c = lax.axis_index("c")
        s = lax.axis_index("s")
        pltpu.sync_copy(neg_hbm, acc_v)
        nbase = c * HALF

        for h in range(2):
            pltpu.sync_copy(src_hbm.at[pl.ds(s * 80 + h * 40, 40)], sidx)
            pltpu.sync_copy(dst_hbm.at[pl.ds(s * 80 + h * 40, 40)], didx)

            def grp(gi, carry):
                gs = [pltpu.async_copy(tab_hbm.at[sidx.at[gi * G4 + j]],
                                       rows_v.at[pl.ds(j * CH, CH)], gsem)
                      for j in range(G4)]
                for d in gs:
                    d.wait()
                for j in range(G4):
                    for g in range(CH // 16):
                        dv = didx[gi * G4 + j, pl.ds(g * 16, 16)]
                        dl = dv - nbase
                        ok = (dl >= 0) & (dl < HALF)
                        idx16 = jnp.where(ok, dl, HALF)
                        for l in range(16):
                            ri = idx16[l]
                            row = rows_v[j * CH + g * 16 + l, :]
                            acc_v[ri, :] = jnp.maximum(acc_v[ri, :], row)
                return carry

            lax.fori_loop(0, 40 // G4, grp, 0)

        pltpu.sync_copy(acc_v, part_hbm.at[c].at[s])
        plsc.subcore_barrier()

        rb = s * (HP // NS)

        def red(p, carry):
            base = rb + p * 64
            ds_ = [pltpu.async_copy(part_hbm.at[c].at[kk].at[pl.ds(base, 64)],
                                    stage.at[kk], gsem) for kk in range(NS)]
            for d in ds_:
                d.wait()

            def row(r, carry2):
                m = stage[0, r, :]
                for kk in range(1, NS):
                    m = jnp.maximum(m, stage[kk, r, :])
                outbuf[r, :] = m
                return carry2

            lax.fori_loop(0, 64, row, 0)
            pltpu.sync_copy(outbuf, out_hbm.at[c].at[pl.ds(base, 64)])
            return carry

        lax.fori_loop(0, (HP // NS) // 64, red, 0)

    return k(src2, dst2, table, neg_rows)


def _tc_prep(x, cnt0, cnt1, nm_W1, nm_b1, nm_W2, nm_b2, g1_W):
    BN = 1000
    grid = N // BN

    def body(x_ref, c0_ref, c1_ref, w1_ref, b1_ref, w2_ref, b2_ref, g1_ref,
             y_ref, xw_ref, dinv_ref, cnt_ref):
        xb = x_ref[...]
        h = jax.nn.relu(
            jnp.dot(xb, w1_ref[...], preferred_element_type=f32) + b1_ref[...])
        h = jnp.dot(h, w2_ref[...], preferred_element_type=f32) + b2_ref[...]
        xw = jnp.dot(h, g1_ref[...], preferred_element_type=f32)
        cnt = c0_ref[...] + c1_ref[...]
        dinv = lax.rsqrt(cnt + 2.0)
        y_ref[...] = dinv * xw
        xw_ref[...] = xw
        dinv_ref[...] = dinv
        cnt_ref[...] = cnt

    outs = pl.pallas_call(
        body,
        grid=(grid,),
        in_specs=[
            pl.BlockSpec((BN, 128), lambda i: (i, 0)),
            pl.BlockSpec((BN, 16), lambda i: (i, 0)),
            pl.BlockSpec((BN, 16), lambda i: (i, 0)),
            pl.BlockSpec((128, 128), lambda i: (0, 0)),
            pl.BlockSpec((1, 128), lambda i: (0, 0)),
            pl.BlockSpec((128, 128), lambda i: (0, 0)),
            pl.BlockSpec((1, 128), lambda i: (0, 0)),
            pl.BlockSpec((128, 16), lambda i: (0, 0)),
        ],
        out_specs=[pl.BlockSpec((BN, 16), lambda i: (i, 0))] * 4,
        out_shape=[jax.ShapeDtypeStruct((N, 16), f32)] * 4,
    )(x, cnt0, cnt1, nm_W1, nm_b1.reshape(1, 128), nm_W2,
      nm_b2.reshape(1, 128), g1_W)
    return outs


def _tc_x1(acc0, acc1, dinv, xw, g1_b, root_W, root_b):
    BN = 1000

    def body(a0_ref, a1_ref, dinv_ref, xw_ref, b_ref, rw_ref, rb_ref,
             x1_ref, x1r_ref):
        dinv = dinv_ref[...]
        x1 = dinv * (a0_ref[...] + a1_ref[...]) \
            + 2.0 * dinv * dinv * xw_ref[...] + b_ref[...]
        x1_ref[...] = x1
        x1r_ref[...] = jnp.dot(x1, rw_ref[...],
                               preferred_element_type=f32) + rb_ref[...]

    return pl.pallas_call(
        body,
        grid=(N // BN,),
        in_specs=[
            pl.BlockSpec((BN, 16), lambda i: (i, 0)),
            pl.BlockSpec((BN, 16), lambda i: (i, 0)),
            pl.BlockSpec((BN, 16), lambda i: (i, 0)),
            pl.BlockSpec((BN, 16), lambda i: (i, 0)),
            pl.BlockSpec((1, 16), lambda i: (0, 0)),
            pl.BlockSpec((16, 16), lambda i: (0, 0)),
            pl.BlockSpec((1, 16), lambda i: (0, 0)),
        ],
        out_specs=[pl.BlockSpec((BN, 16), lambda i: (i, 0))] * 2,
        out_shape=[jax.ShapeDtypeStruct((N, 16), f32)] * 2,
    )(acc0, acc1, dinv, xw, g1_b.reshape(1, 16), root_W, root_b.reshape(1, 16))


def _tc_msg(x1g128, ea_p, em_W1, em_b1, em_W2, em_b2, nn_W1, nn_b1,
            W2r, B2r):
    BE = 2048
    grid = EP // BE

    def body(xg_ref, ea_ref, w1_ref, b1_ref, w2_ref, b2_ref, nw1_ref, nb1_ref,
             w_ref, b_ref, msg_ref):
        ea = ea_ref[...]
        hh = jax.nn.relu(
            jnp.dot(ea, w1_ref[...], preferred_element_type=f32) + b1_ref[...])
        ee = jnp.dot(hh, w2_ref[...], preferred_element_type=f32) + b2_ref[...]
        av = jax.nn.relu(
            jnp.dot(ee, nw1_ref[...], preferred_element_type=f32) + nb1_ref[...])
        x128 = xg_ref[...]
        xg = jnp.concatenate(
            [x128[:, 16 * p:16 * (p + 1)] for p in range(8)], axis=0)
        q = (xg[:, :, None] * av[:, None, :]).reshape(BE, 512)
        msg = (jnp.dot(q, w_ref[...], preferred_element_type=f32)
               + jnp.dot(xg, b_ref[...], preferred_element_type=f32))
        msg_ref[...] = jnp.concatenate(
            [msg[p * (BE // 8):(p + 1) * (BE // 8), :] for p in range(8)], axis=1)

    return pl.pallas_call(
        body,
        grid=(grid,),
        in_specs=[
            pl.BlockSpec((BE // 8, 128), lambda i: (i, 0)),
            pl.BlockSpec((BE, 16), lambda i: (i, 0)),
            pl.BlockSpec((16, 128), lambda i: (0, 0)),
            pl.BlockSpec((1, 128), lambda i: (0, 0)),
            pl.BlockSpec((128, 16), lambda i: (0, 0)),
            pl.BlockSpec((1, 16), lambda i: (0, 0)),
            pl.BlockSpec((16, 32), lambda i: (0, 0)),
            pl.BlockSpec((1, 32), lambda i: (0, 0)),
            pl.BlockSpec((512, 16), lambda i: (0, 0)),
            pl.BlockSpec((16, 16), lambda i: (0, 0)),
        ],
        out_specs=pl.BlockSpec((BE // 8, 128), lambda i: (i, 0)),
        out_shape=jax.ShapeDtypeStruct((EP // 8, 128), f32),
    )(x1g128, ea_p, em_W1, em_b1.reshape(1, 128), em_W2,
      em_b2.reshape(1, 16), nn_W1, nn_b1.reshape(1, 32), W2r, B2r)


def _tc_x2(s0, s1, cnt, x1root, dinv, g2_W):
    BN = 1000

    def body(s0_ref, s1_ref, cnt_ref, x1r_ref, dinv_ref, g2_ref,
             x2_ref, u_ref, c0_ref):
        s = s0_ref[...] + s1_ref[...]
        x2 = jnp.tanh(s / jnp.maximum(cnt_ref[...], 1.0) + x1r_ref[...])
        xw2 = jnp.dot(x2, g2_ref[...], preferred_element_type=f32)
        dinv = dinv_ref[...]
        x2_ref[...] = x2
        u_ref[...] = dinv * xw2
        c0_ref[...] = 2.0 * dinv * xw2

    return pl.pallas_call(
        body,
        grid=(N // BN,),
        in_specs=[
            pl.BlockSpec((BN, 16), lambda i: (i, 0)),
            pl.BlockSpec((BN, 16), lambda i: (i, 0)),
            pl.BlockSpec((BN, 16), lambda i: (i, 0)),
            pl.BlockSpec((BN, 16), lambda i: (i, 0)),
            pl.BlockSpec((BN, 16), lambda i: (i, 0)),
            pl.BlockSpec((16, 16), lambda i: (0, 0)),
        ],
        out_specs=[pl.BlockSpec((BN, 16), lambda i: (i, 0))] * 3,
        out_shape=[jax.ShapeDtypeStruct((N, 16), f32)] * 3,
    )(s0, s1, cnt, x1root, dinv, g2_W)


def _tc_x3(maxp2, c0, dinv, g2_b):
    def body(mp_ref, c0_ref, dinv_ref, b_ref, x3_ref):
        m0 = mp_ref[pl.ds(0, HALF), :]
        m1 = mp_ref[pl.ds(HP, HALF), :]
        m = jnp.concatenate([m0, m1], axis=0)
        x3_ref[...] = dinv_ref[...] * jnp.maximum(m, c0_ref[...]) + b_ref[...]

    return pl.pallas_call(
        body,
        out_shape=jax.ShapeDtypeStruct((N, 16), f32),
    )(maxp2, c0, dinv, g2_b.reshape(1, 16))


def _tc_final(x1, x2, x3, batch1, scales, orientation,
              gate_W1, gate_b1, gate_W2, gate_b2,
              m_W1, m_b1, m_W2, m_b2, m_W3, m_b3):
    def body(x1_ref, x2_ref, x3_ref, b_ref, sc_ref, or_ref,
             gw1_ref, gb1_ref, gw2_ref, gb2_ref,
             mw1_ref, mb1_ref, mw2_ref, mb2_ref, mw3_ref, mb3_ref, out_ref):
        xc = jnp.concatenate([x1_ref[...], x2_ref[...], x3_ref[...]], axis=1)
        g1 = jax.nn.relu(
            jnp.dot(xc, gw1_ref[...], preferred_element_type=f32) + gb1_ref[...])
        gate = jnp.dot(g1, gw2_ref[...], preferred_element_type=f32) + gb2_ref[...]
        b = b_ref[...]
        gid = jax.lax.broadcasted_iota(i32, (1, 8), 1)
        mask = (b == gid)
        gm = jnp.where(mask, gate, -3e38)
        gmax = jnp.max(gm, axis=0, keepdims=True)
        ev = jnp.where(mask, jnp.exp(gate - gmax), 0.0)
        den = jnp.sum(ev, axis=0, keepdims=True)
        w = ev / (den + 1e-16)
        pooled = lax.dot_general(w, xc, (((0,), (0,)), ((), ())),
                                 preferred_element_type=f32)
        feats = jnp.concatenate(
            [pooled, jnp.log(sc_ref[...] + 1e-5), or_ref[...]], axis=1)
        o = jax.nn.relu(
            jnp.dot(feats, mw1_ref[...], preferred_element_type=f32) + mb1_ref[...])
        o = jax.nn.relu(
            jnp.dot(o, mw2_ref[...], preferred_element_type=f32) + mb2_ref[...])
        out_ref[...] = jnp.dot(o, mw3_ref[...],
                               preferred_element_type=f32) + mb3_ref[...]

    return pl.pallas_call(
        body,
        out_shape=jax.ShapeDtypeStruct((8, 8), f32),
    )(x1, x2, x3, batch1, scales, orientation,
      gate_W1, gate_b1.reshape(1, 256), gate_W2, gate_b2.reshape(1, 1),
      m_W1, m_b1.reshape(1, 16), m_W2, m_b2.reshape(1, 8),
      m_W3, m_b3.reshape(1, 8))


def kernel(x, edge_index, edge_attr, batch, scales, orientation,
           nm_W1, nm_b1, nm_W2, nm_b2,
           em_W1, em_b1, em_W2, em_b2,
           g1_W, g1_b,
           nn_W1, nn_b1, nn_W2, nn_b2,
           root_W, root_b,
           g2_W, g2_b,
           gate_W1, gate_b1, gate_W2, gate_b2,
           m_W1, m_b1, m_W2, m_b2, m_W3, m_b3):
    src = edge_index[0]
    dst = edge_index[1]
    pad = EP - E
    src_p = jnp.concatenate([src, jnp.zeros((pad,), i32)])
    dst_p = jnp.concatenate([dst, jnp.full((pad,), N, i32)])
    src2 = src_p.reshape(EP // CH, CH)
    dst2 = dst_p.reshape(EP // CH, CH)
    srcq = src_p.reshape(EP // 2048, 8, 256).transpose(0, 2, 1) \
        .reshape(EP // CH, CH)
    dstq = dst_p.reshape(EP // 2048, 8, 256).transpose(0, 2, 1) \
        .reshape(EP // CH, CH)
    ea_p = jnp.concatenate([edge_attr, jnp.zeros((pad, 16), f32)], axis=0)
    ones_rows = jnp.ones((CH, 16), f32)
    zeros_acc = jnp.zeros((NP, 16), f32)
    neg_rows = jnp.full((HP, 16), -3e38, f32)

    cntp = _sc_count(dst2, ones_rows, zeros_acc)[:, :N]
    y, xw, dinv, cnt = _tc_prep(x, cntp[0], cntp[1],
                                nm_W1, nm_b1, nm_W2, nm_b2, g1_W)
    accp = _sc_gather_scatter(src2, dst2, y, zeros_acc)[:, :N]
    x1, x1root = _tc_x1(accp[0], accp[1], dinv, xw, g1_b, root_W, root_b)
    x1g = _sc_gather_rows(srcq, x1)
    W2r = nn_W2.reshape(32, 16, 16).transpose(1, 0, 2).reshape(512, 16)
    B2r = nn_b2.reshape(16, 16)
    msg128 = _tc_msg(x1g.reshape(EP // 8, 128), ea_p,
                     em_W1, em_b1, em_W2, em_b2, nn_W1, nn_b1, W2r, B2r)
    sp = _sc_scatter_rows(dstq, msg128.reshape(EP, 16),
                          zeros_acc)[:, :N]
    x2, u, c0 = _tc_x2(sp[0], sp[1], cnt, x1root, dinv, g2_W)
    maxp, _unused_partials = _sc_scatter_max(src2, dst2, u, neg_rows)
    x3 = _tc_x3(maxp.reshape(NC * HP, 16), c0, dinv, g2_b)
    return _tc_final(x1, x2, x3, batch.reshape(N, 1).astype(i32),
                     scales, orientation,
                     gate_W1, gate_b1, gate_W2, gate_b2,
                     m_W1, m_b1, m_W2, m_b2, m_W3, m_b3)


# speedup vs baseline: 8.8187x; 1.7208x over previous
import functools

import jax
import jax.numpy as jnp
from jax import lax
from jax.experimental import pallas as pl
from jax.experimental.pallas import tpu as pltpu
from jax.experimental.pallas import tpu_sc as plsc

f32 = jnp.float32
i32 = jnp.int32

N = 10000
E = 160000
EP = 163840
CH = 128
NC = 2
NS = 16
NW = NC * NS
NP = 10240
HALF = N // 2
HP = 5120

_mesh = lambda: plsc.VectorSubcoreMesh(core_axis_name="c", subcore_axis_name="s")


G4 = 4

def _sc_count(dst2, ones_rows, zeros_acc):
    @functools.partial(
        pl.kernel,
        out_type=jax.ShapeDtypeStruct((NC, NP, 16), f32),
        mesh=_mesh(),
        compiler_params=pltpu.CompilerParams(use_tc_tiling_on_sc=False),
        scratch_types=[
            pltpu.VMEM((EP // NW // CH, CH), i32),
            pltpu.VMEM((CH, 16), f32),
            pltpu.VMEM_SHARED((NC, NP, 16), f32),
            pltpu.SemaphoreType.DMA,
        ],
    )
    def k(dst_hbm, ones_hbm, zeros_hbm, out_hbm, didx, ones_v, acc, ssem):
        c = lax.axis_index("c")
        s = lax.axis_index("s")
        r0 = NP // NS
        pltpu.sync_copy(zeros_hbm.at[pl.ds(s * r0, r0)],
                        acc.at[c].at[pl.ds(s * r0, r0)])
        pltpu.sync_copy(ones_hbm, ones_v)
        wid = c * NS + s
        nch = EP // NW // CH
        pltpu.sync_copy(dst_hbm.at[pl.ds(wid * nch, nch)], didx)
        plsc.subcore_barrier()

        def grp(gi, carry):
            ds_ = [pltpu.async_copy(ones_v, acc.at[c].at[didx.at[gi * 8 + j]],
                                    ssem, add=True) for j in range(8)]
            for d in ds_:
                d.wait()
            return carry

        lax.fori_loop(0, nch // 8, grp, 0)
        plsc.subcore_barrier()
        pltpu.sync_copy(acc.at[c].at[pl.ds(s * r0, r0)],
                        out_hbm.at[c].at[pl.ds(s * r0, r0)])

    return k(dst2, ones_rows, zeros_acc)


def _sc_gather_scatter(src2, dst2, table, zeros_acc):
    @functools.partial(
        pl.kernel,
        out_type=jax.ShapeDtypeStruct((NC, NP, 16), f32),
        mesh=_mesh(),
        compiler_params=pltpu.CompilerParams(use_tc_tiling_on_sc=False),
        scratch_types=[
            pltpu.VMEM((EP // NW // CH, CH), i32),
            pltpu.VMEM((EP // NW // CH, CH), i32),
            pltpu.VMEM((2, G4 * CH, 16), f32),
            pltpu.VMEM_SHARED((NC, NP, 16), f32),
            pltpu.SemaphoreType.DMA,
            pltpu.SemaphoreType.DMA,
            pltpu.SemaphoreType.DMA,
            pltpu.SemaphoreType.DMA,
        ],
    )
    def k(src_hbm, dst_hbm, tab_hbm, zeros_hbm, out_hbm,
          sidx, didx, rows_v, acc, gsem0, gsem1, ssem0, ssem1):
        c = lax.axis_index("c")
        s = lax.axis_index("s")
        gsem = [gsem0, gsem1]
        ssem = [ssem0, ssem1]
        r0 = NP // NS
        pltpu.sync_copy(zeros_hbm.at[pl.ds(s * r0, r0)],
                        acc.at[c].at[pl.ds(s * r0, r0)])
        wid = c * NS + s
        nch = EP // NW // CH
        ng = nch // G4
        pltpu.sync_copy(src_hbm.at[pl.ds(wid * nch, nch)], sidx)
        pltpu.sync_copy(dst_hbm.at[pl.ds(wid * nch, nch)], didx)
        plsc.subcore_barrier()

        def fire_gather(gi, b):
            for j in range(G4):
                pltpu.async_copy(tab_hbm.at[sidx.at[gi * G4 + j]],
                                 rows_v.at[b].at[pl.ds(j * CH, CH)], gsem[b])

        def wait_gather(b):
            for j in range(G4):
                pltpu.make_async_copy(
                    tab_hbm.at[sidx.at[0]],
                    rows_v.at[b].at[pl.ds(j * CH, CH)], gsem[b]).wait()

        def fire_scatter(gi, b):
            for j in range(G4):
                pltpu.async_copy(rows_v.at[b].at[pl.ds(j * CH, CH)],
                                 acc.at[c].at[didx.at[gi * G4 + j]],
                                 ssem[b], add=True)

        def wait_scatter(b):
            for j in range(G4):
                pltpu.make_async_copy(
                    rows_v.at[b].at[pl.ds(j * CH, CH)],
                    acc.at[c].at[didx.at[0]], ssem[b]).wait()

        fire_gather(0, 0)

        def outer(kk, carry):
            for b in range(2):
                gi = 2 * kk + b

                @pl.when(gi > 0)
                def _():
                    wait_scatter(1 - b)

                wait_gather(b)

                @pl.when(gi + 1 < ng)
                def _():
                    fire_gather(gi + 1, 1 - b)

                fire_scatter(gi, b)
            return carry

        lax.fori_loop(0, ng // 2, outer, 0)
        wait_scatter(1)
        plsc.subcore_barrier()
        pltpu.sync_copy(acc.at[c].at[pl.ds(s * r0, r0)],
                        out_hbm.at[c].at[pl.ds(s * r0, r0)])

    return k(src2, dst2, table, zeros_acc)


def _sc_scatter_rows(dst2, rows, zeros_acc):
    @functools.partial(
        pl.kernel,
        out_type=jax.ShapeDtypeStruct((NC, NP, 16), f32),
        mesh=_mesh(),
        compiler_params=pltpu.CompilerParams(use_tc_tiling_on_sc=False),
        scratch_types=[
            pltpu.VMEM((EP // NW // CH, CH), i32),
            pltpu.VMEM((2, G4 * CH, 16), f32),
            pltpu.VMEM_SHARED((NC, NP, 16), f32),
            pltpu.SemaphoreType.DMA,
            pltpu.SemaphoreType.DMA,
            pltpu.SemaphoreType.DMA,
            pltpu.SemaphoreType.DMA,
        ],
    )
    def k(dst_hbm, rows_hbm, zeros_hbm, out_hbm, didx, rows_v, acc,
          lsem0, lsem1, ssem0, ssem1):
        c = lax.axis_index("c")
        s = lax.axis_index("s")
        lsem = [lsem0, lsem1]
        ssem = [ssem0, ssem1]
        r0 = NP // NS
        pltpu.sync_copy(zeros_hbm.at[pl.ds(s * r0, r0)],
                        acc.at[c].at[pl.ds(s * r0, r0)])
        wid = c * NS + s
        nch = EP // NW // CH
        ng = nch // G4
        pltpu.sync_copy(dst_hbm.at[pl.ds(wid * nch, nch)], didx)
        plsc.subcore_barrier()
        ebase = wid * (EP // NW)

        def fire_load(gi, b):
            pltpu.async_copy(
                rows_hbm.at[pl.ds(ebase + gi * G4 * CH, G4 * CH)],
                rows_v.at[b], lsem[b])

        def wait_load(b):
            pltpu.make_async_copy(
                rows_hbm.at[pl.ds(ebase, G4 * CH)], rows_v.at[b],
                lsem[b]).wait()

        def fire_scatter(gi, b):
            for j in range(G4):
                pltpu.async_copy(rows_v.at[b].at[pl.ds(j * CH, CH)],
                                 acc.at[c].at[didx.at[gi * G4 + j]],
                                 ssem[b], add=True)

        def wait_scatter(b):
            for j in range(G4):
                pltpu.make_async_copy(
                    rows_v.at[b].at[pl.ds(j * CH, CH)],
                    acc.at[c].at[didx.at[0]], ssem[b]).wait()

        fire_load(0, 0)

        def outer(kk, carry):
            for b in range(2):
                gi = 2 * kk + b

                @pl.when(gi > 0)
                def _():
                    wait_scatter(1 - b)

                @pl.when(gi + 1 < ng)
                def _():
                    fire_load(gi + 1, 1 - b)

                wait_load(b)
                fire_scatter(gi, b)
            return carry

        lax.fori_loop(0, ng // 2, outer, 0)
        wait_scatter(1)
        plsc.subcore_barrier()
        pltpu.sync_copy(acc.at[c].at[pl.ds(s * r0, r0)],
                        out_hbm.at[c].at[pl.ds(s * r0, r0)])

    return k(dst2, rows, zeros_acc)


def _sc_gather_rows(src2, table):
    @functools.partial(
        pl.kernel,
        out_type=jax.ShapeDtypeStruct((EP, 16), f32),
        mesh=_mesh(),
        compiler_params=pltpu.CompilerParams(use_tc_tiling_on_sc=False),
        scratch_types=[
            pltpu.VMEM((EP // NW // CH, CH), i32),
            pltpu.VMEM((2, G4 * CH, 16), f32),
            pltpu.SemaphoreType.DMA,
            pltpu.SemaphoreType.DMA,
        ],
    )
    def k(src_hbm, tab_hbm, out_hbm, sidx, rows_v, gsem0, gsem1):
        c = lax.axis_index("c")
        s = lax.axis_index("s")
        gsem = [gsem0, gsem1]
        wid = c * NS + s
        nch = EP // NW // CH
        ng = nch // G4
        pltpu.sync_copy(src_hbm.at[pl.ds(wid * nch, nch)], sidx)
        ebase = wid * (EP // NW)

        def fire_gather(gi, b):
            for j in range(G4):
                pltpu.async_copy(tab_hbm.at[sidx.at[gi * G4 + j]],
                                 rows_v.at[b].at[pl.ds(j * CH, CH)], gsem[b])

        def wait_gather(b):
            for j in range(G4):
                pltpu.make_async_copy(
                    tab_hbm.at[sidx.at[0]],
                    rows_v.at[b].at[pl.ds(j * CH, CH)], gsem[b]).wait()

        fire_gather(0, 0)

        def outer(kk, carry):
            for b in range(2):
                gi = 2 * kk + b
                wait_gather(b)

                @pl.when(gi + 1 < ng)
                def _():
                    fire_gather(gi + 1, 1 - b)

                pltpu.sync_copy(
                    rows_v.at[b],
                    out_hbm.at[pl.ds(ebase + gi * G4 * CH, G4 * CH)])
            return carry

        lax.fori_loop(0, ng // 2, outer, 0)

    return k(src2, table)


GM = 2


def _sc_scatter_max(src2, dst2, table, neg_rows):
    @functools.partial(
        pl.kernel,
        out_type=[jax.ShapeDtypeStruct((NC, HP, 16), f32),
                  jax.ShapeDtypeStruct((NC, NS, HP, 16), f32)],
        mesh=_mesh(),
        compiler_params=pltpu.CompilerParams(use_tc_tiling_on_sc=False),
        scratch_types=[
            pltpu.VMEM((40, CH), i32),
            pltpu.VMEM((40, CH), i32),
            pltpu.VMEM((2, GM * CH, 16), f32),
            pltpu.VMEM((HP, 16), f32),
            pltpu.VMEM((NS, 64, 16), f32),
            pltpu.VMEM((64, 16), f32),
            pltpu.SemaphoreType.DMA,
            pltpu.SemaphoreType.DMA,
        ],
    )
    def k(src_hbm, dst_hbm, tab_hbm, neg_hbm, out_hbm, part_hbm,
          sidx, didx, rows_v, acc_v, stage, outbuf, gsem0, gsem1):
        c = lax.axis_index("c")
        s = lax.axis_index("s")
        gsem = [gsem0, gsem1]
        pltpu.sync_copy(neg_hbm, acc_v)
        nbase = c * HALF

        def fire_gather(gi, b):
            for j in range(GM):
                pltpu.async_copy(tab_hbm.at[sidx.at[gi * GM + j]],
                                 rows_v.at[b].at[pl.ds(j * CH, CH)], gsem[b])

        def wait_gather(b):
            for j in range(GM):
                pltpu.make_async_copy(
                    tab_hbm.at[sidx.at[0]],
                    rows_v.at[b].at[pl.ds(j * CH, CH)], gsem[b]).wait()

        ng = 40 // GM
        for h in range(2):
            pltpu.sync_copy(src_hbm.at[pl.ds(s * 80 + h * 40, 40)], sidx)
            pltpu.sync_copy(dst_hbm.at[pl.ds(s * 80 + h * 40, 40)], didx)
            fire_gather(0, 0)

            def outer(kk, carry):
                for b in range(2):
                    gi = 2 * kk + b
                    wait_gather(b)

                    @pl.when(gi + 1 < ng)
                    def _():
                        fire_gather(gi + 1, 1 - b)

                    for j in range(GM):
                        for g in range(CH // 16):
                            dv = didx[gi * G4 + j, pl.ds(g * 16, 16)]
                            dl = dv - nbase
                            ok = (dl >= 0) & (dl < HALF)
                            idx16 = jnp.where(ok, dl, HALF)
                            for l in range(16):
                                ri = idx16[l]
                                row = rows_v[b, j * CH + g * 16 + l, :]
                                acc_v[ri, :] = jnp.maximum(acc_v[ri, :], row)
                return carry

            lax.fori_loop(0, ng // 2, outer, 0)

        pltpu.sync_copy(acc_v, part_hbm.at[c].at[s])
        plsc.subcore_barrier()

        rb = s * (HP // NS)

        def red(p, carry):
            base = rb + p * 64
            ds_ = [pltpu.async_copy(part_hbm.at[c].at[kk].at[pl.ds(base, 64)],
                                    stage.at[kk], gsem0) for kk in range(NS)]
            for d in ds_:
                d.wait()

            def row(r, carry2):
                m = stage[0, r, :]
                for kk in range(1, NS):
                    m = jnp.maximum(m, stage[kk, r, :])
                outbuf[r, :] = m
                return carry2

            lax.fori_loop(0, 64, row, 0)
            pltpu.sync_copy(outbuf, out_hbm.at[c].at[pl.ds(base, 64)])
            return carry

        lax.fori_loop(0, (HP // NS) // 64, red, 0)

    return k(src2, dst2, table, neg_rows)


def _tc_prep(x, cnt0, cnt1, nm_W1, nm_b1, nm_W2, nm_b2, g1_W):
    BN = 1000
    grid = N // BN

    def body(x_ref, c0_ref, c1_ref, w1_ref, b1_ref, wf_ref, bf_ref,
             y_ref, xw_ref, dinv_ref, cnt_ref):
        xb = x_ref[...]
        h = jax.nn.relu(
            jnp.dot(xb, w1_ref[...], preferred_element_type=f32) + b1_ref[...])
        xw = jnp.dot(h, wf_ref[...], preferred_element_type=f32) + bf_ref[...]
        cnt = c0_ref[...] + c1_ref[...]
        dinv = lax.rsqrt(cnt + 2.0)
        y_ref[...] = dinv * xw
        xw_ref[...] = xw
        dinv_ref[...] = dinv
        cnt_ref[...] = cnt

    outs = pl.pallas_call(
        body,
        grid=(grid,),
        in_specs=[
            pl.BlockSpec((BN, 128), lambda i: (i, 0)),
            pl.BlockSpec((BN, 16), lambda i: (i, 0)),
            pl.BlockSpec((BN, 16), lambda i: (i, 0)),
            pl.BlockSpec((128, 128), lambda i: (0, 0)),
            pl.BlockSpec((1, 128), lambda i: (0, 0)),
            pl.BlockSpec((128, 16), lambda i: (0, 0)),
            pl.BlockSpec((1, 16), lambda i: (0, 0)),
        ],
        out_specs=[pl.BlockSpec((BN, 16), lambda i: (i, 0))] * 4,
        out_shape=[jax.ShapeDtypeStruct((N, 16), f32)] * 4,
    )(x, cnt0, cnt1, nm_W1, nm_b1.reshape(1, 128),
      jnp.dot(nm_W2, g1_W), jnp.dot(nm_b2, g1_W).reshape(1, 16))
    return outs


def _tc_x1(acc0, acc1, dinv, xw, g1_b, root_W, root_b):
    BN = 1000

    def body(a0_ref, a1_ref, dinv_ref, xw_ref, b_ref, rw_ref, rb_ref,
             x1_ref, x1r_ref):
        dinv = dinv_ref[...]
        x1 = dinv * (a0_ref[...] + a1_ref[...]) \
            + 2.0 * dinv * dinv * xw_ref[...] + b_ref[...]
        x1_ref[...] = x1
        x1r_ref[...] = jnp.dot(x1, rw_ref[...],
                               preferred_element_type=f32) + rb_ref[...]

    return pl.pallas_call(
        body,
        grid=(N // BN,),
        in_specs=[
            pl.BlockSpec((BN, 16), lambda i: (i, 0)),
            pl.BlockSpec((BN, 16), lambda i: (i, 0)),
            pl.BlockSpec((BN, 16), lambda i: (i, 0)),
            pl.BlockSpec((BN, 16), lambda i: (i, 0)),
            pl.BlockSpec((1, 16), lambda i: (0, 0)),
            pl.BlockSpec((16, 16), lambda i: (0, 0)),
            pl.BlockSpec((1, 16), lambda i: (0, 0)),
        ],
        out_specs=[pl.BlockSpec((BN, 16), lambda i: (i, 0))] * 2,
        out_shape=[jax.ShapeDtypeStruct((N, 16), f32)] * 2,
    )(acc0, acc1, dinv, xw, g1_b.reshape(1, 16), root_W, root_b.reshape(1, 16))


def _tc_msg(x1g128, ea_p, em_W1, em_b1, em_W2, em_b2, nn_W1, nn_b1,
            W2r, B2r, RX, RA):
    BE = 2048
    grid = EP // BE

    def body(xg_ref, ea_ref, w1_ref, b1_ref, w2_ref, b2_ref, nw1_ref, nb1_ref,
             w_ref, b_ref, rx_ref, ra_ref, msg_ref):
        ea = ea_ref[...]
        hh = jax.nn.relu(
            jnp.dot(ea, w1_ref[...], preferred_element_type=f32) + b1_ref[...])
        ee = jnp.dot(hh, w2_ref[...], preferred_element_type=f32) + b2_ref[...]
        av = jax.nn.relu(
            jnp.dot(ee, nw1_ref[...], preferred_element_type=f32) + nb1_ref[...])
        x128 = xg_ref[...]
        xg = jnp.concatenate(
            [x128[:, 16 * p:16 * (p + 1)] for p in range(8)], axis=0)
        q = (jnp.dot(xg, rx_ref[...], preferred_element_type=f32)
             * jnp.dot(av, ra_ref[...], preferred_element_type=f32))
        msg = (jnp.dot(q, w_ref[...], preferred_element_type=f32)
               + jnp.dot(xg, b_ref[...], preferred_element_type=f32))
        msg_ref[...] = jnp.concatenate(
            [msg[p * (BE // 8):(p + 1) * (BE // 8), :] for p in range(8)], axis=1)

    return pl.pallas_call(
        body,
        grid=(grid,),
        in_specs=[
            pl.BlockSpec((BE // 8, 128), lambda i: (i, 0)),
            pl.BlockSpec((BE, 16), lambda i: (i, 0)),
            pl.BlockSpec((16, 128), lambda i: (0, 0)),
            pl.BlockSpec((1, 128), lambda i: (0, 0)),
            pl.BlockSpec((128, 16), lambda i: (0, 0)),
            pl.BlockSpec((1, 16), lambda i: (0, 0)),
            pl.BlockSpec((16, 32), lambda i: (0, 0)),
            pl.BlockSpec((1, 32), lambda i: (0, 0)),
            pl.BlockSpec((512, 16), lambda i: (0, 0)),
            pl.BlockSpec((16, 16), lambda i: (0, 0)),
            pl.BlockSpec((16, 512), lambda i: (0, 0)),
            pl.BlockSpec((32, 512), lambda i: (0, 0)),
        ],
        out_specs=pl.BlockSpec((BE // 8, 128), lambda i: (i, 0)),
        out_shape=jax.ShapeDtypeStruct((EP // 8, 128), f32),
    )(x1g128, ea_p, em_W1, em_b1.reshape(1, 128), em_W2,
      em_b2.reshape(1, 16), nn_W1, nn_b1.reshape(1, 32), W2r, B2r, RX, RA)


def _tc_x2(s0, s1, cnt, x1root, dinv, g2_W):
    BN = 1000

    def body(s0_ref, s1_ref, cnt_ref, x1r_ref, dinv_ref, g2_ref,
             x2_ref, u_ref, c0_ref):
        s = s0_ref[...] + s1_ref[...]
        x2 = jnp.tanh(s / jnp.maximum(cnt_ref[...], 1.0) + x1r_ref[...])
        xw2 = jnp.dot(x2, g2_ref[...], preferred_element_type=f32)
        dinv = dinv_ref[...]
        x2_ref[...] = x2
        u_ref[...] = dinv * xw2
        c0_ref[...] = 2.0 * dinv * xw2

    return pl.pallas_call(
        body,
        grid=(N // BN,),
        in_specs=[
            pl.BlockSpec((BN, 16), lambda i: (i, 0)),
            pl.BlockSpec((BN, 16), lambda i: (i, 0)),
            pl.BlockSpec((BN, 16), lambda i: (i, 0)),
            pl.BlockSpec((BN, 16), lambda i: (i, 0)),
            pl.BlockSpec((BN, 16), lambda i: (i, 0)),
            pl.BlockSpec((16, 16), lambda i: (0, 0)),
        ],
        out_specs=[pl.BlockSpec((BN, 16), lambda i: (i, 0))] * 3,
        out_shape=[jax.ShapeDtypeStruct((N, 16), f32)] * 3,
    )(s0, s1, cnt, x1root, dinv, g2_W)


def _tc_final(x1, x2, maxp2, c0, dinv, g2_b, batch1, scales, orientation,
              gate_W1, gate_b1, gate_W2, gate_b2,
              m_W1, m_b1, m_W2, m_b2, m_W3, m_b3):
    def body(x1_ref, x2_ref, mp_ref, c0_ref, dinv_ref, g2b_ref,
             b_ref, sc_ref, or_ref,
             gw1_ref, gb1_ref, gw2_ref, gb2_ref,
             mw1_ref, mb1_ref, mw2_ref, mb2_ref, mw3_ref, mb3_ref, out_ref):
        m = jnp.concatenate(
            [mp_ref[pl.ds(0, HALF), :], mp_ref[pl.ds(HP, HALF), :]], axis=0)
        x3 = dinv_ref[...] * jnp.maximum(m, c0_ref[...]) + g2b_ref[...]
        xc = jnp.concatenate([x1_ref[...], x2_ref[...], x3], axis=1)
        g1 = jax.nn.relu(
            jnp.dot(xc, gw1_ref[...], preferred_element_type=f32) + gb1_ref[...])
        gate = jnp.dot(g1, gw2_ref[...], preferred_element_type=f32) + gb2_ref[...]
        b = b_ref[...]
        gid = jax.lax.broadcasted_iota(i32, (1, 8), 1)
        mask = (b == gid)
        gm = jnp.where(mask, gate, -3e38)
        gmax = jnp.max(gm, axis=0, keepdims=True)
        ev = jnp.where(mask, jnp.exp(gate - gmax), 0.0)
        den = jnp.sum(ev, axis=0, keepdims=True)
        w = ev / (den + 1e-16)
        pooled = lax.dot_general(w, xc, (((0,), (0,)), ((), ())),
                                 preferred_element_type=f32)
        feats = jnp.concatenate(
            [pooled, jnp.log(sc_ref[...] + 1e-5), or_ref[...]], axis=1)
        o = jax.nn.relu(
            jnp.dot(feats, mw1_ref[...], preferred_element_type=f32) + mb1_ref[...])
        o = jax.nn.relu(
            jnp.dot(o, mw2_ref[...], preferred_element_type=f32) + mb2_ref[...])
        out_ref[...] = jnp.dot(o, mw3_ref[...],
                               preferred_element_type=f32) + mb3_ref[...]

    return pl.pallas_call(
        body,
        out_shape=jax.ShapeDtypeStruct((8, 8), f32),
    )(x1, x2, maxp2, c0, dinv, g2_b.reshape(1, 16), batch1, scales,
      orientation,
      gate_W1, gate_b1.reshape(1, 256), gate_W2, gate_b2.reshape(1, 1),
      m_W1, m_b1.reshape(1, 16), m_W2, m_b2.reshape(1, 8),
      m_W3, m_b3.reshape(1, 8))


def kernel(x, edge_index, edge_attr, batch, scales, orientation,
           nm_W1, nm_b1, nm_W2, nm_b2,
           em_W1, em_b1, em_W2, em_b2,
           g1_W, g1_b,
           nn_W1, nn_b1, nn_W2, nn_b2,
           root_W, root_b,
           g2_W, g2_b,
           gate_W1, gate_b1, gate_W2, gate_b2,
           m_W1, m_b1, m_W2, m_b2, m_W3, m_b3):
    src = edge_index[0]
    dst = edge_index[1]
    pad = EP - E
    src_p = jnp.concatenate([src, jnp.zeros((pad,), i32)])
    dst_p = jnp.concatenate([dst, jnp.full((pad,), N, i32)])
    src2 = src_p.reshape(EP // CH, CH)
    dst2 = dst_p.reshape(EP // CH, CH)
    srcq = src_p.reshape(EP // 2048, 8, 256).transpose(0, 2, 1) \
        .reshape(EP // CH, CH)
    dstq = dst_p.reshape(EP // 2048, 8, 256).transpose(0, 2, 1) \
        .reshape(EP // CH, CH)
    ea_p = jnp.concatenate([edge_attr, jnp.zeros((pad, 16), f32)], axis=0)
    ones_rows = jnp.ones((CH, 16), f32)
    zeros_acc = jnp.zeros((NP, 16), f32)
    neg_rows = jnp.full((HP, 16), -3e38, f32)

    cntp = _sc_count(dst2, ones_rows, zeros_acc)[:, :N]
    y, xw, dinv, cnt = _tc_prep(x, cntp[0], cntp[1],
                                nm_W1, nm_b1, nm_W2, nm_b2, g1_W)
    accp = _sc_gather_scatter(src2, dst2, y, zeros_acc)[:, :N]
    x1, x1root = _tc_x1(accp[0], accp[1], dinv, xw, g1_b, root_W, root_b)
    x1g = _sc_gather_rows(srcq, x1)
    W2r = nn_W2.reshape(32, 16, 16).transpose(1, 0, 2).reshape(512, 16)
    B2r = nn_b2.reshape(16, 16)
    RX = jnp.repeat(jnp.eye(16, dtype=f32), 32, axis=1)
    RA = jnp.tile(jnp.eye(32, dtype=f32), (1, 16))
    msg128 = _tc_msg(x1g.reshape(EP // 8, 128), ea_p,
                     em_W1, em_b1, em_W2, em_b2, nn_W1, nn_b1, W2r, B2r, RX, RA)
    sp = _sc_scatter_rows(dstq, msg128.reshape(EP, 16),
                          zeros_acc)[:, :N]
    x2, u, c0 = _tc_x2(sp[0], sp[1], cnt, x1root, dinv, g2_W)
    maxp, _unused_partials = _sc_scatter_max(src2, dst2, u, neg_rows)
    return _tc_final(x1, x2, maxp.reshape(NC * HP, 16), c0, dinv, g2_b,
                     batch.reshape(N, 1).astype(i32), scales, orientation,
                     gate_W1, gate_b1, gate_W2, gate_b2,
                     m_W1, m_b1, m_W2, m_b2, m_W3, m_b3)
